# baseline (speedup 1.0000x reference)
import functools

import jax
import jax.numpy as jnp
from jax import lax
from jax.experimental import pallas as pl
from jax.experimental.pallas import tpu as pltpu
from jax.experimental.pallas import tpu_sc as plsc

VOCAB = 100000
DIM = 128
B = 1024
L = 200
TOTAL = B * L

NC = 2
NS = 16
NW = NC * NS
PER_W = TOTAL // NW
CH = 128
NCH = PER_W // CH


NBUF = 5


LOOKAHEAD = 3


def _sc_gather_body(ids_hbm, table_hbm, out_hbm, idx_v, *bufs_and_sems):
    bufs = bufs_and_sems[:NBUF]
    gsems = bufs_and_sems[NBUF : 2 * NBUF]
    wsems = bufs_and_sems[2 * NBUF :]
    wid = lax.axis_index("s") * NC + lax.axis_index("c")
    base = wid * PER_W

    pltpu.sync_copy(ids_hbm.at[pl.ds(base, PER_W)], idx_v)

    def start_gather(ch, b):
        pltpu.async_copy(
            table_hbm.at[idx_v.at[pl.ds(ch * CH, CH)]], bufs[b], gsems[b]
        )

    def wait_gather(ch, b):
        pltpu.make_async_copy(
            table_hbm.at[idx_v.at[pl.ds(ch * CH, CH)]], bufs[b], gsems[b]
        ).wait()

    def start_write(ch, b):
        pltpu.async_copy(
            bufs[b], out_hbm.at[pl.ds(base + ch * CH, CH), :], wsems[b]
        )

    def wait_write(ch, b):
        pltpu.make_async_copy(
            bufs[b], out_hbm.at[pl.ds(base + ch * CH, CH), :], wsems[b]
        ).wait()

    for b in range(LOOKAHEAD):
        start_gather(b, b)

    for ch in range(NBUF - LOOKAHEAD):
        wait_gather(ch, ch % NBUF)
        start_write(ch, ch % NBUF)
        start_gather(ch + LOOKAHEAD, (ch + LOOKAHEAD) % NBUF)

    @pl.loop(NBUF - LOOKAHEAD, NCH - LOOKAHEAD, step=NBUF)
    def _(c):
        for k in range(NBUF):
            ch = c + k
            b = (NBUF - LOOKAHEAD + k) % NBUF
            bj = (NBUF + k) % NBUF
            wait_gather(ch, b)
            start_write(ch, b)
            wait_write(ch + LOOKAHEAD - NBUF, bj)
            start_gather(ch + LOOKAHEAD, bj)

    for ch in range(NCH - LOOKAHEAD, NCH):
        wait_gather(ch, ch % NBUF)
        start_write(ch, ch % NBUF)

    for ch in range(NCH - NBUF, NCH):
        wait_write(ch, ch % NBUF)


_sc_gather = pl.kernel(
    _sc_gather_body,
    out_type=jax.ShapeDtypeStruct((TOTAL, DIM), jnp.float32),
    mesh=plsc.VectorSubcoreMesh(core_axis_name="c", subcore_axis_name="s"),
    scratch_types=(
        [pltpu.VMEM((PER_W,), jnp.int32)]
        + [pltpu.VMEM((CH, DIM), jnp.float32) for _ in range(NBUF)]
        + [pltpu.SemaphoreType.DMA for _ in range(2 * NBUF)]
    ),
)


ROWS_BLK = 10000
GRID = VOCAB // ROWS_BLK


def _ln_body(x_ref, g_ref, b_ref, o_ref):
    x = x_ref[...]
    ones = jnp.ones((DIM, 1), dtype=jnp.float32)
    s = jax.lax.dot(x, ones, preferred_element_type=jnp.float32)
    sq = jax.lax.dot(x * x, ones, preferred_element_type=jnp.float32)
    mean = s * (1.0 / DIM)
    var = sq * (1.0 / DIM) - mean * mean
    o_ref[...] = (x - mean) * lax.rsqrt(var + 1e-5) * g_ref[...] + b_ref[...]


def _tc_layernorm(rows, gamma, beta):
    return pl.pallas_call(
        _ln_body,
        grid=(GRID,),
        in_specs=[
            pl.BlockSpec((ROWS_BLK, DIM), lambda i: (i, 0)),
            pl.BlockSpec((1, DIM), lambda i: (0, 0)),
            pl.BlockSpec((1, DIM), lambda i: (0, 0)),
        ],
        out_specs=pl.BlockSpec((ROWS_BLK, DIM), lambda i: (i, 0)),
        out_shape=jax.ShapeDtypeStruct((VOCAB, DIM), jnp.float32),
    )(rows, gamma.reshape(1, DIM), beta.reshape(1, DIM))


def kernel(ids, table, gamma, beta):
    ids_flat = ids.reshape(-1).astype(jnp.int32)
    normed = _tc_layernorm(table, gamma, beta)
    out = _sc_gather(ids_flat, normed)
    return out.reshape(B, L, DIM)


# speedup vs baseline: 1.0195x; 1.0195x over previous
import functools

import jax
import jax.numpy as jnp
from jax import lax
from jax.experimental import pallas as pl
from jax.experimental.pallas import tpu as pltpu
from jax.experimental.pallas import tpu_sc as plsc

VOCAB = 100000
DIM = 128
B = 1024
L = 200
TOTAL = B * L

NC = 2
NS = 16
NW = NC * NS
PER_W = TOTAL // NW
CH = 128
NCH = PER_W // CH


NBUF = 5


LOOKAHEAD = 3


def _sc_gather_body(ids_hbm, table_hbm, out_hbm, idx_v, *bufs_and_sems):
    bufs = bufs_and_sems[:NBUF]
    gsems = bufs_and_sems[NBUF : 2 * NBUF]
    wsems = bufs_and_sems[2 * NBUF :]
    wid = lax.axis_index("s") * NC + lax.axis_index("c")
    base = wid * PER_W

    pltpu.sync_copy(ids_hbm.at[wid], idx_v)

    def start_gather(ch, b):
        pltpu.async_copy(
            table_hbm.at[idx_v.at[pl.ds(ch * CH, CH)]], bufs[b], gsems[b]
        )

    def wait_gather(ch, b):
        pltpu.make_async_copy(
            table_hbm.at[idx_v.at[pl.ds(ch * CH, CH)]], bufs[b], gsems[b]
        ).wait()

    def start_write(ch, b):
        pltpu.async_copy(
            bufs[b], out_hbm.at[pl.ds(base + ch * CH, CH), :], wsems[b]
        )

    def wait_write(ch, b):
        pltpu.make_async_copy(
            bufs[b], out_hbm.at[pl.ds(base + ch * CH, CH), :], wsems[b]
        ).wait()

    for b in range(LOOKAHEAD):
        start_gather(b, b)

    for ch in range(NBUF - LOOKAHEAD):
        wait_gather(ch, ch % NBUF)
        start_write(ch, ch % NBUF)
        start_gather(ch + LOOKAHEAD, (ch + LOOKAHEAD) % NBUF)

    @pl.loop(NBUF - LOOKAHEAD, NCH - LOOKAHEAD, step=NBUF)
    def _(c):
        for k in range(NBUF):
            ch = c + k
            b = (NBUF - LOOKAHEAD + k) % NBUF
            bj = (NBUF + k) % NBUF
            wait_gather(ch, b)
            start_write(ch, b)
            wait_write(ch + LOOKAHEAD - NBUF, bj)
            start_gather(ch + LOOKAHEAD, bj)

    for ch in range(NCH - LOOKAHEAD, NCH):
        wait_gather(ch, ch % NBUF)
        start_write(ch, ch % NBUF)

    for ch in range(NCH - NBUF, NCH):
        wait_write(ch, ch % NBUF)


_sc_gather = pl.kernel(
    _sc_gather_body,
    out_type=jax.ShapeDtypeStruct((TOTAL, DIM), jnp.float32),
    mesh=plsc.VectorSubcoreMesh(core_axis_name="c", subcore_axis_name="s"),
    scratch_types=(
        [pltpu.VMEM((PER_W,), jnp.int32)]
        + [pltpu.VMEM((CH, DIM), jnp.float32) for _ in range(NBUF)]
        + [pltpu.SemaphoreType.DMA for _ in range(2 * NBUF)]
    ),
)


ROWS_BLK = 10000
GRID = VOCAB // ROWS_BLK


def _ln_body(x_ref, g_ref, b_ref, o_ref):
    x = x_ref[...]
    mean = jnp.mean(x, axis=-1, keepdims=True)
    xc = x - mean
    var = jnp.mean(xc * xc, axis=-1, keepdims=True)
    o_ref[...] = xc * lax.rsqrt(var + 1e-5) * g_ref[...] + b_ref[...]


def _tc_layernorm(rows, gamma, beta):
    return pl.pallas_call(
        _ln_body,
        grid=(GRID,),
        in_specs=[
            pl.BlockSpec((ROWS_BLK, DIM), lambda i: (i, 0)),
            pl.BlockSpec((1, DIM), lambda i: (0, 0)),
            pl.BlockSpec((1, DIM), lambda i: (0, 0)),
        ],
        out_specs=pl.BlockSpec((ROWS_BLK, DIM), lambda i: (i, 0)),
        out_shape=jax.ShapeDtypeStruct((VOCAB, DIM), jnp.float32),
    )(rows, gamma.reshape(1, DIM), beta.reshape(1, DIM))


def kernel(ids, table, gamma, beta):
    ids_rows = ids.reshape(NW, PER_W).astype(jnp.int32)
    normed = _tc_layernorm(table, gamma, beta)
    out = _sc_gather(ids_rows, normed)
    return out.reshape(B, L, DIM)
